# baseline (speedup 1.0000x reference)
import functools

import jax
import jax.numpy as jnp
from jax import lax
from jax.experimental import pallas as pl
from jax.experimental.pallas import tpu as pltpu
from jax.experimental.pallas import tpu_sc as plsc

N = 50000
E = 800000
G = 2048
D = 64
DH = 32
BINS = 50
SAMP = 1000

NC = 2
NS = 16
CH = 128
SAC = 512
EP = 819200
NP = 53248
NW = 50176
NA = NW + SAC
GA = G + SAC

f32 = jnp.float32
i32 = jnp.int32

_MESH = plsc.VectorSubcoreMesh(core_axis_name="c", subcore_axis_name="s")

_DEGCH = EP // (NC * NS * CH)
_CNTCH = NP // (NC * NS * CH)

_WCHUNKS = ((0, 1024), (1024, 1024), (2048, 1024), (3072, 64))
_RPT = 3136


def _degcnt_body(dst_hbm, batch_hbm, ones_hbm, zeros_hbm, deg_out, cnt_out,
                 didx_all, cidx_all, ones_v, deg_acc, cnt_acc, psem):
    c = lax.axis_index("c")
    s = lax.axis_index("s")

    erow = (c * NS + s) * _DEGCH
    nrow = (c * NS + s) * _CNTCH
    pltpu.async_copy(dst_hbm.at[pl.ds(erow, _DEGCH), :], didx_all, psem)
    pltpu.async_copy(batch_hbm.at[pl.ds(nrow, _CNTCH), :], cidx_all, psem)
    pltpu.sync_copy(ones_hbm, ones_v)

    for off, sz in _WCHUNKS:
        pltpu.sync_copy(zeros_hbm.at[pl.ds(0, sz), :],
                        deg_acc.at[pl.ds(s * _RPT + off, sz), :])
    pltpu.sync_copy(zeros_hbm.at[pl.ds(0, G // NS), :],
                    cnt_acc.at[pl.ds(s * (G // NS), G // NS), :])
    plsc.subcore_barrier()

    pltpu.make_async_copy(dst_hbm.at[pl.ds(erow, _DEGCH), :],
                          didx_all, psem).wait()
    pltpu.make_async_copy(batch_hbm.at[pl.ds(nrow, _CNTCH), :],
                          cidx_all, psem).wait()

    K = 8

    def deg_group(i, carry):
        for j in range(K):
            pltpu.async_copy(ones_v, deg_acc.at[didx_all.at[i * K + j]],
                             psem, add=True)
        for j in range(K):
            pltpu.make_async_copy(ones_v, deg_acc.at[didx_all.at[0]],
                                  psem).wait()
        return carry

    lax.fori_loop(0, _DEGCH // K, deg_group, 0)

    for j in range(_CNTCH):
        pltpu.async_copy(ones_v, cnt_acc.at[cidx_all.at[j]], psem, add=True)
    for j in range(_CNTCH):
        pltpu.make_async_copy(ones_v, cnt_acc.at[cidx_all.at[0]],
                              psem).wait()
    plsc.subcore_barrier()

    for off, sz in _WCHUNKS:
        pltpu.sync_copy(deg_acc.at[pl.ds(s * _RPT + off, sz), :],
                        deg_out.at[pl.ds(s * _RPT + off, sz),
                                   pl.ds(c * 8, 8)])
    gpt = G // NS
    pltpu.sync_copy(cnt_acc.at[pl.ds(s * gpt, gpt), :],
                    cnt_out.at[pl.ds(s * gpt, gpt), pl.ds(c * 8, 8)])


def _degcnt_call(dst_2d, batch_2d, ones8, zeros8):
    fn = functools.partial(
        pl.kernel,
        mesh=_MESH,
        compiler_params=pltpu.CompilerParams(use_tc_tiling_on_sc=False),
        out_type=[jax.ShapeDtypeStruct((NW, 128), f32),
                  jax.ShapeDtypeStruct((G, 128), f32)],
        scratch_types=[
            pltpu.VMEM((_DEGCH, CH), i32),
            pltpu.VMEM((_CNTCH, CH), i32),
            pltpu.VMEM((CH, 8), f32),
            pltpu.VMEM_SHARED((NA, 8), f32),
            pltpu.VMEM_SHARED((GA, 8), f32),
            pltpu.SemaphoreType.DMA,
        ],
    )(_degcnt_body)
    return fn(dst_2d, batch_2d, ones8, zeros8)


NB = 4


def _conv_body(table_hbm, src4_hbm, dst_hbm, zeros_hbm, y_out,
               sidx, didx, rows, acc, *sems):
    c = lax.axis_index("c")
    s = lax.axis_index("s")
    isems = sems[0:NB]
    isemd = sems[NB:2 * NB]
    gsem = sems[2 * NB:3 * NB]
    ssem = sems[3 * NB:4 * NB]

    for off, sz in _WCHUNKS:
        pltpu.sync_copy(zeros_hbm.at[pl.ds(0, sz), :],
                        acc.at[pl.ds(s * _RPT + off, sz), :])
    plsc.subcore_barrier()

    ept = EP // NS
    nsteps = ept // CH
    grow = (c * EP + s * ept) // CH
    drow = (s * ept) // CH

    def load_idx(st, b):
        pltpu.async_copy(src4_hbm.at[pl.ds(grow + st, 1), :],
                         sidx.at[b], isems[b])
        pltpu.async_copy(dst_hbm.at[pl.ds(drow + st, 1), :],
                         didx.at[b], isemd[b])

    def drain_sidx(b):
        pltpu.make_async_copy(src4_hbm.at[pl.ds(grow, 1), :],
                              sidx.at[b], isems[b]).wait()

    def drain_didx(b):
        pltpu.make_async_copy(dst_hbm.at[pl.ds(drow, 1), :],
                              didx.at[b], isemd[b]).wait()

    def fire_gather(b):
        pltpu.async_copy(table_hbm.at[sidx.at[b, 0]], rows.at[b], gsem[b])

    def drain_gather(b):
        pltpu.make_async_copy(table_hbm.at[sidx.at[b, 0]],
                              rows.at[b], gsem[b]).wait()

    def fire_scatter(b):
        pltpu.async_copy(rows.at[b], acc.at[didx.at[b, 0]],
                         ssem[b], add=True)

    def drain_scatter(b):
        pltpu.make_async_copy(rows.at[b], acc.at[didx.at[b, 0]],
                              ssem[b]).wait()

    for b in range(NB):
        load_idx(b, b)
    drain_sidx(0)
    fire_gather(0)

    def group(i, carry):
        for b in range(NB):
            st = NB * i + b
            bn = (b + 1) % NB
            if b == NB - 1:
                drain_scatter(bn)
            else:
                @pl.when(i > 0)
                def _():
                    drain_scatter(bn)
            drain_sidx(bn)
            fire_gather(bn)
            drain_gather(b)
            drain_didx(b)
            fire_scatter(b)
            load_idx(st + NB, b)
        return carry

    lax.fori_loop(0, nsteps // NB, group, 0)

    for b in range(1, NB):
        drain_scatter(b)
    drain_gather(0)
    drain_didx(0)
    for b in range(1, NB):
        drain_sidx(b)
        drain_didx(b)
    plsc.subcore_barrier()

    for off, sz in _WCHUNKS:
        pltpu.sync_copy(acc.at[pl.ds(s * _RPT + off, sz), :],
                        y_out.at[pl.ds(s * _RPT + off, sz),
                                 pl.ds(c * DH, DH)])


def _conv_call(s_view, src4_2d, dst_2d, zerosDH):
    fn = functools.partial(
        pl.kernel,
        mesh=_MESH,
        compiler_params=pltpu.CompilerParams(use_tc_tiling_on_sc=False),
        out_type=jax.ShapeDtypeStruct((NW, 128), f32),
        scratch_types=[
            pltpu.VMEM((NB, 1, CH), i32),
            pltpu.VMEM((NB, 1, CH), i32),
            pltpu.VMEM((NB, CH, DH), f32),
            pltpu.VMEM_SHARED((NA, DH), f32),
        ] + [pltpu.SemaphoreType.DMA] * (4 * NB),
    )(_conv_body)
    return fn(s_view, src4_2d, dst_2d, zerosDH)


def _pool_body(x_hbm, batch_hbm, zeros_hbm, p_out,
               bidx_v, rows_v, acc):
    c = lax.axis_index("c")
    s = lax.axis_index("s")
    gpt = G // NS

    pltpu.sync_copy(zeros_hbm.at[pl.ds(0, gpt), :],
                    acc.at[pl.ds(s * gpt, gpt), :])
    plsc.subcore_barrier()

    npt = NP // NS
    nbase = s * npt

    def chunk(i, carry):
        base = nbase + i * CH
        pltpu.sync_copy(batch_hbm.at[pl.ds(base, CH)], bidx_v)
        pltpu.sync_copy(x_hbm.at[pl.ds(base, CH), pl.ds(c * DH, DH)],
                        rows_v)
        pltpu.sync_copy(rows_v, acc.at[bidx_v], add=True)
        return carry

    lax.fori_loop(0, npt // CH, chunk, 0)
    plsc.subcore_barrier()

    pltpu.sync_copy(acc.at[pl.ds(s * gpt, gpt), :],
                    p_out.at[pl.ds(s * gpt, gpt), pl.ds(c * DH, DH)])


def _pool_call(x_view, batch_p, zerosDH):
    fn = functools.partial(
        pl.kernel,
        mesh=_MESH,
        compiler_params=pltpu.CompilerParams(use_tc_tiling_on_sc=False),
        out_type=jax.ShapeDtypeStruct((G, 128), f32),
        scratch_types=[
            pltpu.VMEM((CH,), i32),
            pltpu.VMEM((CH, DH), f32),
            pltpu.VMEM_SHARED((GA, DH), f32),
        ],
    )(_pool_body)
    return fn(x_view, batch_p, zerosDH)


TB = 1024
TCOMB = 160


def _enc_body(xcat_ref, xh_ref, xs_ref, deg_ref, T_ref, Wh_ref, Ws_ref,
              Wp_ref, W1_ref, s1_ref):
    cdims = (((0,), (0,)), ((), ()))
    xcat = xcat_ref[...]
    iota_t = lax.broadcasted_iota(i32, (TB, TCOMB), 1)
    oh = jnp.zeros((TB, TCOMB), f32)
    for j in range(5):
        oh = oh + (iota_t == xcat[j, :][:, None]).astype(f32)
    emb = jnp.dot(oh, T_ref[...], preferred_element_type=f32)
    h = (emb
         + lax.dot_general(xh_ref[...], Wh_ref[...], cdims,
                           preferred_element_type=f32)
         + lax.dot_general(xs_ref[...], Ws_ref[...], cdims,
                           preferred_element_type=f32))
    h = jnp.maximum(jnp.dot(h, Wp_ref[...], preferred_element_type=f32), 0.0)

    dblk = deg_ref[...]
    dinv = lax.rsqrt(dblk[:, 0:1] + dblk[:, 8:9] + 1.0)
    s1 = dinv * jnp.dot(h, W1_ref[...], preferred_element_type=f32)
    s1_ref[:, 0:D] = s1


def _enc_call(xcatT, x_histT, x_sampleT, deg, T, W_hist, W_samp,
              W_proj, W1):
    return pl.pallas_call(
        _enc_body,
        grid=(NW // TB,),
        in_specs=[
            pl.BlockSpec((8, TB), lambda i: (0, i)),
            pl.BlockSpec((BINS, TB), lambda i: (0, i)),
            pl.BlockSpec((SAMP, TB), lambda i: (0, i)),
            pl.BlockSpec((TB, 128), lambda i: (i, 0)),
            pl.BlockSpec((TCOMB, D), lambda i: (0, 0)),
            pl.BlockSpec((BINS, D), lambda i: (0, 0)),
            pl.BlockSpec((SAMP, D), lambda i: (0, 0)),
            pl.BlockSpec((D, D), lambda i: (0, 0)),
            pl.BlockSpec((D, D), lambda i: (0, 0)),
        ],
        out_specs=pl.BlockSpec((TB, 128), lambda i: (i, 0)),
        out_shape=jax.ShapeDtypeStruct((NW, 128), f32),
    )(xcatT, x_histT, x_sampleT, deg, T, W_hist, W_samp, W_proj, W1)


MB = 2048
_NBLK = (NW + MB - 1) // MB


def _mid_body(y_ref, s_ref, deg_ref, Wn_ref, out_ref):
    dblk = deg_ref[...]
    dinv = lax.rsqrt(dblk[:, 0:1] + dblk[:, 8:9] + 1.0)
    x = jnp.maximum(dinv * (y_ref[:, 0:D] + s_ref[:, 0:D]), 0.0)
    sn = dinv * jnp.dot(x, Wn_ref[...], preferred_element_type=f32)
    out_ref[:, 0:D] = sn


def _fin_body(y_ref, s_ref, deg_ref, out_ref):
    dblk = deg_ref[...]
    dinv = lax.rsqrt(dblk[:, 0:1] + dblk[:, 8:9] + 1.0)
    x = dinv * (y_ref[:, 0:D] + s_ref[:, 0:D])
    out_ref[:, 0:D] = x


def _mid_call(y, s, deg, Wn):
    spec = pl.BlockSpec((MB, 128), lambda i: (i, 0))
    return pl.pallas_call(
        _mid_body,
        grid=(_NBLK,),
        in_specs=[spec, spec, spec,
                  pl.BlockSpec((D, D), lambda i: (0, 0))],
        out_specs=spec,
        out_shape=jax.ShapeDtypeStruct((NW, 128), f32),
    )(y, s, deg, Wn)


def _fin_call(y, s, deg):
    spec = pl.BlockSpec((MB, 128), lambda i: (i, 0))
    return pl.pallas_call(
        _fin_body,
        grid=(_NBLK,),
        in_specs=[spec, spec, spec],
        out_specs=spec,
        out_shape=jax.ShapeDtypeStruct((NP, 128), f32),
    )(y, s, deg)


def _head_body(pool_ref, cnt_ref, wh_ref, out_ref):
    pool = pool_ref[:, 0:D]
    cntb = cnt_ref[...]
    cnt = cntb[:, 0:1] + cntb[:, 8:9]
    pooled = pool / jnp.maximum(cnt, 1.0)
    wh = wh_ref[0:1, :]
    z = jnp.sum(pooled * wh, axis=1, keepdims=True)
    sig = 1.0 / (1.0 + jnp.exp(-z))
    out_ref[...] = jnp.broadcast_to(sig.reshape(1, G), (8, G))


def _head_call(pool, cnt, wh8):
    return pl.pallas_call(
        _head_body,
        grid=(1,),
        in_specs=[
            pl.BlockSpec((G, 128), lambda i: (0, 0)),
            pl.BlockSpec((G, 128), lambda i: (0, 0)),
            pl.BlockSpec((8, D), lambda i: (0, 0)),
        ],
        out_specs=pl.BlockSpec((8, G), lambda i: (0, 0)),
        out_shape=jax.ShapeDtypeStruct((8, G), f32),
    )(pool, cnt, wh8)


def kernel(x_cat, x_hist, x_sample, edge_index, batch,
           emb_type, emb_table, emb_join, emb_col, emb_op,
           W_hist, b_hist, W_samp, b_samp, W_proj, b_proj,
           W1, b1, W2, b2, W3, b3, W_head, b_head):
    src = edge_index[0].astype(i32)
    dst = edge_index[1].astype(i32)
    batch = batch.astype(i32)

    epad = jnp.arange(EP - E, dtype=i32)
    src_p = jnp.concatenate([src, epad % SAC])
    dst_p = jnp.concatenate([dst, NW + (epad % SAC)])
    opad = jnp.arange(512, dtype=i32) % CH
    src4_p = jnp.concatenate([4 * src_p, 4 * src_p + 1, opad])
    src4_2d = src4_p.reshape(-1, CH)
    dst_2d = jnp.concatenate([dst_p, NW + opad]).reshape(-1, CH)
    npad = jnp.arange(NP - N, dtype=i32)
    batch_p = jnp.concatenate([batch, G + (npad % SAC)])
    batch_2d = batch_p.reshape(-1, CH)

    T = jnp.zeros((TCOMB, D), f32)
    T = T.at[0:20].set(emb_type)
    T = T.at[20:45].set(emb_table)
    T = T.at[45:85].set(emb_join)
    T = T.at[85:145].set(emb_col)
    T = T.at[145:155].set(emb_op)
    offs = jnp.array([0, 20, 45, 85, 145], i32)
    xcatT = jnp.concatenate(
        [x_cat.astype(i32).T + offs[:, None],
         jnp.full((3, N), TCOMB - 1, i32)], axis=0)

    ones8 = jnp.ones((CH, 8), f32)
    zeros8 = jnp.zeros((1024, 8), f32)
    zerosDH = jnp.zeros((1024, DH), f32)

    deg, cnt = _degcnt_call(dst_2d, batch_2d, ones8, zeros8)

    s1 = _enc_call(xcatT, x_hist.T, x_sample.T, deg, T,
                   W_hist, W_samp, W_proj, W1)

    y1 = _conv_call(s1.reshape(4 * NW, DH), src4_2d, dst_2d, zerosDH)
    s2 = _mid_call(y1, s1, deg, W2)
    y2 = _conv_call(s2.reshape(4 * NW, DH), src4_2d, dst_2d, zerosDH)
    s3 = _mid_call(y2, s2, deg, W3)
    y3 = _conv_call(s3.reshape(4 * NW, DH), src4_2d, dst_2d, zerosDH)
    x3 = _fin_call(y3, s3, deg)

    pool = _pool_call(x3, batch_p, zerosDH)

    wh8 = jnp.broadcast_to(W_head.reshape(1, D), (8, D))
    out8 = _head_call(pool, cnt, wh8)
    return out8[0].reshape(G, 1)


# speedup vs baseline: 1.0423x; 1.0423x over previous
import functools

import jax
import jax.numpy as jnp
from jax import lax
from jax.experimental import pallas as pl
from jax.experimental.pallas import tpu as pltpu
from jax.experimental.pallas import tpu_sc as plsc

N = 50000
E = 800000
G = 2048
D = 64
DH = 32
BINS = 50
SAMP = 1000

NC = 2
NS = 16
CH = 128
SAC = 512
EP = 819200
NP = 53248
NW = 50176
NA = NW + SAC
GA = G + SAC

f32 = jnp.float32
i32 = jnp.int32

_MESH = plsc.VectorSubcoreMesh(core_axis_name="c", subcore_axis_name="s")

_DEGCH = EP // (NC * NS * CH)
_CNTCH = NP // (NC * NS * CH)

_WCHUNKS = ((0, 1024), (1024, 1024), (2048, 1024), (3072, 64))
_RPT = 3136


def _degcnt_body(dst_hbm, batch_hbm, ones_hbm, zeros_hbm, deg_out, cnt_out,
                 didx_all, cidx_all, ones_v, deg_acc, cnt_acc, psem):
    c = lax.axis_index("c")
    s = lax.axis_index("s")

    erow = (c * NS + s) * _DEGCH
    nrow = (c * NS + s) * _CNTCH
    pltpu.async_copy(dst_hbm.at[pl.ds(erow, _DEGCH), :], didx_all, psem)
    pltpu.async_copy(batch_hbm.at[pl.ds(nrow, _CNTCH), :], cidx_all, psem)
    pltpu.sync_copy(ones_hbm, ones_v)

    for off, sz in _WCHUNKS:
        pltpu.sync_copy(zeros_hbm.at[pl.ds(0, sz), :],
                        deg_acc.at[pl.ds(s * _RPT + off, sz), :])
    pltpu.sync_copy(zeros_hbm.at[pl.ds(0, G // NS), :],
                    cnt_acc.at[pl.ds(s * (G // NS), G // NS), :])
    plsc.subcore_barrier()

    pltpu.make_async_copy(dst_hbm.at[pl.ds(erow, _DEGCH), :],
                          didx_all, psem).wait()
    pltpu.make_async_copy(batch_hbm.at[pl.ds(nrow, _CNTCH), :],
                          cidx_all, psem).wait()

    K = 8

    def deg_group(i, carry):
        for j in range(K):
            pltpu.async_copy(ones_v, deg_acc.at[didx_all.at[i * K + j]],
                             psem, add=True)
        for j in range(K):
            pltpu.make_async_copy(ones_v, deg_acc.at[didx_all.at[0]],
                                  psem).wait()
        return carry

    lax.fori_loop(0, _DEGCH // K, deg_group, 0)

    for j in range(_CNTCH):
        pltpu.async_copy(ones_v, cnt_acc.at[cidx_all.at[j]], psem, add=True)
    for j in range(_CNTCH):
        pltpu.make_async_copy(ones_v, cnt_acc.at[cidx_all.at[0]],
                              psem).wait()
    plsc.subcore_barrier()

    for off, sz in _WCHUNKS:
        pltpu.sync_copy(deg_acc.at[pl.ds(s * _RPT + off, sz), :],
                        deg_out.at[pl.ds(s * _RPT + off, sz),
                                   pl.ds(c * 8, 8)])
    gpt = G // NS
    pltpu.sync_copy(cnt_acc.at[pl.ds(s * gpt, gpt), :],
                    cnt_out.at[pl.ds(s * gpt, gpt), pl.ds(c * 8, 8)])


def _degcnt_call(dst_2d, batch_2d, ones8, zeros8):
    fn = functools.partial(
        pl.kernel,
        mesh=_MESH,
        compiler_params=pltpu.CompilerParams(use_tc_tiling_on_sc=False),
        out_type=[jax.ShapeDtypeStruct((NW, 128), f32),
                  jax.ShapeDtypeStruct((G, 128), f32)],
        scratch_types=[
            pltpu.VMEM((_DEGCH, CH), i32),
            pltpu.VMEM((_CNTCH, CH), i32),
            pltpu.VMEM((CH, 8), f32),
            pltpu.VMEM_SHARED((NA, 8), f32),
            pltpu.VMEM_SHARED((GA, 8), f32),
            pltpu.SemaphoreType.DMA,
        ],
    )(_degcnt_body)
    return fn(dst_2d, batch_2d, ones8, zeros8)


def _conv_body(table_hbm, src4_hbm, dst_hbm, zeros_hbm, y_out,
               sidx, didx, rows, acc,
               isems0, isems1, isemd0, isemd1, gsem0, gsem1, ssem0, ssem1):
    c = lax.axis_index("c")
    s = lax.axis_index("s")
    isems = (isems0, isems1)
    isemd = (isemd0, isemd1)
    gsem = (gsem0, gsem1)
    ssem = (ssem0, ssem1)

    for off, sz in _WCHUNKS:
        pltpu.sync_copy(zeros_hbm.at[pl.ds(0, sz), :],
                        acc.at[pl.ds(s * _RPT + off, sz), :])
    plsc.subcore_barrier()

    ept = EP // NS
    nsteps = ept // (2 * CH)
    grow = (c * EP + s * ept) // CH
    drow = (s * ept) // CH

    def load_idx(st, b):
        pltpu.async_copy(src4_hbm.at[pl.ds(grow + st * 2, 2), :],
                         sidx.at[b], isems[b])
        pltpu.async_copy(dst_hbm.at[pl.ds(drow + st * 2, 2), :],
                         didx.at[b], isemd[b])

    def drain_sidx(b):
        pltpu.make_async_copy(src4_hbm.at[pl.ds(grow, 2), :],
                              sidx.at[b], isems[b]).wait()

    def drain_didx(b):
        pltpu.make_async_copy(dst_hbm.at[pl.ds(drow, 2), :],
                              didx.at[b], isemd[b]).wait()

    def fire_gathers(b):
        for j in range(2):
            pltpu.async_copy(table_hbm.at[sidx.at[b, j]], rows.at[b, j],
                             gsem[b])

    def drain_gathers(b):
        for j in range(2):
            pltpu.make_async_copy(table_hbm.at[sidx.at[b, j]],
                                  rows.at[b, j], gsem[b]).wait()

    def fire_scatters(b):
        for j in range(2):
            pltpu.async_copy(rows.at[b, j], acc.at[didx.at[b, j]],
                             ssem[b], add=True)

    def drain_scatters(b):
        for j in range(2):
            pltpu.make_async_copy(rows.at[b, j], acc.at[didx.at[b, j]],
                                  ssem[b]).wait()

    load_idx(0, 0)
    load_idx(1, 1)
    drain_sidx(0)
    fire_gathers(0)

    def group(i, carry):
        for b in range(2):
            st = 2 * i + b
            nb = 1 - b
            if b == 0:
                @pl.when(i > 0)
                def _():
                    drain_scatters(nb)
            else:
                drain_scatters(nb)
            drain_sidx(nb)
            fire_gathers(nb)
            drain_gathers(b)
            drain_didx(b)
            fire_scatters(b)
            load_idx(st + 2, b)
        return carry

    lax.fori_loop(0, nsteps // 2, group, 0)

    drain_scatters(1)
    drain_gathers(0)
    drain_didx(0)
    drain_sidx(1)
    drain_didx(1)
    plsc.subcore_barrier()

    for off, sz in _WCHUNKS:
        pltpu.sync_copy(acc.at[pl.ds(s * _RPT + off, sz), :],
                        y_out.at[pl.ds(s * _RPT + off, sz),
                                 pl.ds(c * DH, DH)])


def _conv_call(s_view, src4_2d, dst_2d, zerosDH):
    fn = functools.partial(
        pl.kernel,
        mesh=_MESH,
        compiler_params=pltpu.CompilerParams(use_tc_tiling_on_sc=False),
        out_type=jax.ShapeDtypeStruct((NW, 128), f32),
        scratch_types=[
            pltpu.VMEM((2, 2, CH), i32),
            pltpu.VMEM((2, 2, CH), i32),
            pltpu.VMEM((2, 2, CH, DH), f32),
            pltpu.VMEM_SHARED((NA, DH), f32),
        ] + [pltpu.SemaphoreType.DMA] * 8,
    )(_conv_body)
    return fn(s_view, src4_2d, dst_2d, zerosDH)


def _pool_body(x_hbm, batch_hbm, zeros_hbm, p_out,
               bidx_v, rows_v, acc):
    c = lax.axis_index("c")
    s = lax.axis_index("s")
    gpt = G // NS

    pltpu.sync_copy(zeros_hbm.at[pl.ds(0, gpt), :],
                    acc.at[pl.ds(s * gpt, gpt), :])
    plsc.subcore_barrier()

    npt = NP // NS
    nbase = s * npt

    def chunk(i, carry):
        base = nbase + i * CH
        pltpu.sync_copy(batch_hbm.at[pl.ds(base, CH)], bidx_v)
        pltpu.sync_copy(x_hbm.at[pl.ds(base, CH), pl.ds(c * DH, DH)],
                        rows_v)
        pltpu.sync_copy(rows_v, acc.at[bidx_v], add=True)
        return carry

    lax.fori_loop(0, npt // CH, chunk, 0)
    plsc.subcore_barrier()

    pltpu.sync_copy(acc.at[pl.ds(s * gpt, gpt), :],
                    p_out.at[pl.ds(s * gpt, gpt), pl.ds(c * DH, DH)])


def _pool_call(x_view, batch_p, zerosDH):
    fn = functools.partial(
        pl.kernel,
        mesh=_MESH,
        compiler_params=pltpu.CompilerParams(use_tc_tiling_on_sc=False),
        out_type=jax.ShapeDtypeStruct((G, 128), f32),
        scratch_types=[
            pltpu.VMEM((CH,), i32),
            pltpu.VMEM((CH, DH), f32),
            pltpu.VMEM_SHARED((GA, DH), f32),
        ],
    )(_pool_body)
    return fn(x_view, batch_p, zerosDH)


TB = 1024
TCOMB = 160


def _enc_body(xcat_ref, xh_ref, xs_ref, deg_ref, T_ref, Wh_ref, Ws_ref,
              Wp_ref, W1_ref, s1_ref):
    cdims = (((0,), (0,)), ((), ()))
    xcat = xcat_ref[...]
    iota_t = lax.broadcasted_iota(i32, (TB, TCOMB), 1)
    oh = jnp.zeros((TB, TCOMB), f32)
    for j in range(5):
        oh = oh + (iota_t == xcat[j, :][:, None]).astype(f32)
    emb = jnp.dot(oh, T_ref[...], preferred_element_type=f32)
    h = (emb
         + lax.dot_general(xh_ref[...], Wh_ref[...], cdims,
                           preferred_element_type=f32)
         + lax.dot_general(xs_ref[...], Ws_ref[...], cdims,
                           preferred_element_type=f32))
    h = jnp.maximum(jnp.dot(h, Wp_ref[...], preferred_element_type=f32), 0.0)

    dblk = deg_ref[...]
    dinv = lax.rsqrt(dblk[:, 0:1] + dblk[:, 8:9] + 1.0)
    s1 = dinv * jnp.dot(h, W1_ref[...], preferred_element_type=f32)
    s1_ref[:, 0:D] = s1


def _enc_call(xcatT, x_histT, x_sampleT, deg, T, W_hist, W_samp,
              W_proj, W1):
    return pl.pallas_call(
        _enc_body,
        grid=(NW // TB,),
        in_specs=[
            pl.BlockSpec((8, TB), lambda i: (0, i)),
            pl.BlockSpec((BINS, TB), lambda i: (0, i)),
            pl.BlockSpec((SAMP, TB), lambda i: (0, i)),
            pl.BlockSpec((TB, 128), lambda i: (i, 0)),
            pl.BlockSpec((TCOMB, D), lambda i: (0, 0)),
            pl.BlockSpec((BINS, D), lambda i: (0, 0)),
            pl.BlockSpec((SAMP, D), lambda i: (0, 0)),
            pl.BlockSpec((D, D), lambda i: (0, 0)),
            pl.BlockSpec((D, D), lambda i: (0, 0)),
        ],
        out_specs=pl.BlockSpec((TB, 128), lambda i: (i, 0)),
        out_shape=jax.ShapeDtypeStruct((NW, 128), f32),
    )(xcatT, x_histT, x_sampleT, deg, T, W_hist, W_samp, W_proj, W1)


MB = 2048
_NBLK = (NW + MB - 1) // MB


def _mid_body(y_ref, s_ref, deg_ref, Wn_ref, out_ref):
    dblk = deg_ref[...]
    dinv = lax.rsqrt(dblk[:, 0:1] + dblk[:, 8:9] + 1.0)
    x = jnp.maximum(dinv * (y_ref[:, 0:D] + s_ref[:, 0:D]), 0.0)
    sn = dinv * jnp.dot(x, Wn_ref[...], preferred_element_type=f32)
    out_ref[:, 0:D] = sn


def _fin_body(y_ref, s_ref, deg_ref, out_ref):
    dblk = deg_ref[...]
    dinv = lax.rsqrt(dblk[:, 0:1] + dblk[:, 8:9] + 1.0)
    x = dinv * (y_ref[:, 0:D] + s_ref[:, 0:D])
    out_ref[:, 0:D] = x


def _mid_call(y, s, deg, Wn):
    spec = pl.BlockSpec((MB, 128), lambda i: (i, 0))
    return pl.pallas_call(
        _mid_body,
        grid=(_NBLK,),
        in_specs=[spec, spec, spec,
                  pl.BlockSpec((D, D), lambda i: (0, 0))],
        out_specs=spec,
        out_shape=jax.ShapeDtypeStruct((NW, 128), f32),
    )(y, s, deg, Wn)


def _fin_call(y, s, deg):
    spec = pl.BlockSpec((MB, 128), lambda i: (i, 0))
    return pl.pallas_call(
        _fin_body,
        grid=(_NBLK,),
        in_specs=[spec, spec, spec],
        out_specs=spec,
        out_shape=jax.ShapeDtypeStruct((NP, 128), f32),
    )(y, s, deg)


def _head_body(pool_ref, cnt_ref, wh_ref, out_ref):
    pool = pool_ref[:, 0:D]
    cntb = cnt_ref[...]
    cnt = cntb[:, 0:1] + cntb[:, 8:9]
    pooled = pool / jnp.maximum(cnt, 1.0)
    wh = wh_ref[0:1, :]
    z = jnp.sum(pooled * wh, axis=1, keepdims=True)
    sig = 1.0 / (1.0 + jnp.exp(-z))
    out_ref[...] = jnp.broadcast_to(sig.reshape(1, G), (8, G))


def _head_call(pool, cnt, wh8):
    return pl.pallas_call(
        _head_body,
        grid=(1,),
        in_specs=[
            pl.BlockSpec((G, 128), lambda i: (0, 0)),
            pl.BlockSpec((G, 128), lambda i: (0, 0)),
            pl.BlockSpec((8, D), lambda i: (0, 0)),
        ],
        out_specs=pl.BlockSpec((8, G), lambda i: (0, 0)),
        out_shape=jax.ShapeDtypeStruct((8, G), f32),
    )(pool, cnt, wh8)


def kernel(x_cat, x_hist, x_sample, edge_index, batch,
           emb_type, emb_table, emb_join, emb_col, emb_op,
           W_hist, b_hist, W_samp, b_samp, W_proj, b_proj,
           W1, b1, W2, b2, W3, b3, W_head, b_head):
    src = edge_index[0].astype(i32)
    dst = edge_index[1].astype(i32)
    batch = batch.astype(i32)

    epad = jnp.arange(EP - E, dtype=i32)
    src_p = jnp.concatenate([src, epad % SAC])
    dst_p = jnp.concatenate([dst, NW + (epad % SAC)])
    opad = jnp.arange(512, dtype=i32) % CH
    src4_p = jnp.concatenate([4 * src_p, 4 * src_p + 1, opad])
    src4_2d = src4_p.reshape(-1, CH)
    dst_2d = jnp.concatenate([dst_p, NW + opad]).reshape(-1, CH)
    npad = jnp.arange(NP - N, dtype=i32)
    batch_p = jnp.concatenate([batch, G + (npad % SAC)])
    batch_2d = batch_p.reshape(-1, CH)

    T = jnp.zeros((TCOMB, D), f32)
    T = T.at[0:20].set(emb_type)
    T = T.at[20:45].set(emb_table)
    T = T.at[45:85].set(emb_join)
    T = T.at[85:145].set(emb_col)
    T = T.at[145:155].set(emb_op)
    offs = jnp.array([0, 20, 45, 85, 145], i32)
    xcatT = jnp.concatenate(
        [x_cat.astype(i32).T + offs[:, None],
         jnp.full((3, N), TCOMB - 1, i32)], axis=0)

    ones8 = jnp.ones((CH, 8), f32)
    zeros8 = jnp.zeros((1024, 8), f32)
    zerosDH = jnp.zeros((1024, DH), f32)

    deg, cnt = _degcnt_call(dst_2d, batch_2d, ones8, zeros8)

    s1 = _enc_call(xcatT, x_hist.T, x_sample.T, deg, T,
                   W_hist, W_samp, W_proj, W1)

    y1 = _conv_call(s1.reshape(4 * NW, DH), src4_2d, dst_2d, zerosDH)
    s2 = _mid_call(y1, s1, deg, W2)
    y2 = _conv_call(s2.reshape(4 * NW, DH), src4_2d, dst_2d, zerosDH)
    s3 = _mid_call(y2, s2, deg, W3)
    y3 = _conv_call(s3.reshape(4 * NW, DH), src4_2d, dst_2d, zerosDH)
    x3 = _fin_call(y3, s3, deg)

    pool = _pool_call(x3, batch_p, zerosDH)

    wh8 = jnp.broadcast_to(W_head.reshape(1, D), (8, D))
    out8 = _head_call(pool, cnt, wh8)
    return out8[0].reshape(G, 1)
